# baseline (speedup 1.0000x reference)
import functools

import jax
import jax.numpy as jnp
from jax import lax
from jax.experimental import pallas as pl
from jax.experimental.pallas import tpu as pltpu
from jax.experimental.pallas import tpu_sc as plsc

B = 16384
N_ROWS = 1000000
MF_D = 8
MLP_D = 32

_E = 2048


def _tc_transform_body(mfu_t, mfi_t, mlu_t, mli_t, eye8, w1u, w1i,
                       out_mfu, out_mfi, out_tu, out_ti):
  f32 = jnp.float32
  dn = (((0,), (0,)), ((), ()))
  out_mfu[...] = lax.dot_general(mfu_t[...], eye8[...], dn,
                                 preferred_element_type=f32)
  out_mfi[...] = lax.dot_general(mfi_t[...], eye8[...], dn,
                                 preferred_element_type=f32)
  out_tu[...] = lax.dot_general(mlu_t[...], w1u[...], dn,
                                preferred_element_type=f32)
  out_ti[...] = lax.dot_general(mli_t[...], w1i[...], dn,
                                preferred_element_type=f32)


def _tc_transform(mf_user_table, mf_item_table, mlp_user_table,
                  mlp_item_table, W1):
  f32 = jnp.float32
  mfu_t = mf_user_table.T
  mfi_t = mf_item_table.T
  mlu_t = mlp_user_table.T
  mli_t = mlp_item_table.T
  eye8 = jnp.eye(MF_D, dtype=f32)
  w1u = W1[:MLP_D]
  w1i = W1[MLP_D:]
  grid = (pl.cdiv(N_ROWS, _E),)

  def col_spec(d):
    return pl.BlockSpec((d, _E), lambda i: (0, i))

  def full_spec(a):
    return pl.BlockSpec(a.shape, lambda i: (0, 0))

  def row_spec(d):
    return pl.BlockSpec((_E, d), lambda i: (i, 0))

  return pl.pallas_call(
      _tc_transform_body,
      grid=grid,
      in_specs=[
          col_spec(MF_D), col_spec(MF_D), col_spec(MLP_D), col_spec(MLP_D),
          full_spec(eye8), full_spec(w1u), full_spec(w1i),
      ],
      out_specs=[row_spec(MF_D), row_spec(MF_D), row_spec(MLP_D),
                 row_spec(MLP_D)],
      out_shape=[
          jax.ShapeDtypeStruct((N_ROWS, MF_D), f32),
          jax.ShapeDtypeStruct((N_ROWS, MF_D), f32),
          jax.ShapeDtypeStruct((N_ROWS, MLP_D), f32),
          jax.ShapeDtypeStruct((N_ROWS, MLP_D), f32),
      ],
  )(mfu_t, mfi_t, mlu_t, mli_t, eye8, w1u, w1i)


def _sc_gather(user, item, mfu_lin, mfi_lin, tu_lin, ti_lin):
  info = plsc.get_sparse_core_info()
  nw = info.num_cores * info.num_subcores
  b_per_w = B // nw
  ch = 128
  n_chunks = b_per_w // ch
  mesh = plsc.VectorSubcoreMesh(core_axis_name="c", subcore_axis_name="s")
  f32 = jnp.float32

  @functools.partial(
      pl.kernel,
      mesh=mesh,
      compiler_params=pltpu.CompilerParams(use_tc_tiling_on_sc=False),
      out_type=[
          jax.ShapeDtypeStruct((B, MF_D), f32),
          jax.ShapeDtypeStruct((B, MF_D), f32),
          jax.ShapeDtypeStruct((B, MLP_D), f32),
          jax.ShapeDtypeStruct((B, MLP_D), f32),
      ],
      scratch_types=[
          pltpu.VMEM((ch,), jnp.int32),
          pltpu.VMEM((ch,), jnp.int32),
          pltpu.VMEM((ch, MF_D), f32),
          pltpu.VMEM((ch, MF_D), f32),
          pltpu.VMEM((ch, MLP_D), f32),
          pltpu.VMEM((ch, MLP_D), f32),
          pltpu.SemaphoreType.DMA,
      ],
  )
  def gather_kernel(user_h, item_h, mfu_h, mfi_h, tu_h, ti_h,
                    out_mfu, out_mfi, out_tu, out_ti,
                    idx_u, idx_i, r_mfu, r_mfi, r_tu, r_ti, sem):
    wid = lax.axis_index("s") * info.num_cores + lax.axis_index("c")
    base = wid * b_per_w
    for c in range(n_chunks):
      off = base + c * ch
      pltpu.sync_copy(user_h.at[pl.ds(off, ch)], idx_u)
      pltpu.sync_copy(item_h.at[pl.ds(off, ch)], idx_i)
      g1 = pltpu.async_copy(mfu_h.at[idx_u], r_mfu, sem)
      g2 = pltpu.async_copy(mfi_h.at[idx_i], r_mfi, sem)
      g3 = pltpu.async_copy(tu_h.at[idx_u], r_tu, sem)
      g4 = pltpu.async_copy(ti_h.at[idx_i], r_ti, sem)
      g1.wait()
      g2.wait()
      g3.wait()
      g4.wait()
      pltpu.sync_copy(r_mfu, out_mfu.at[pl.ds(off, ch)])
      pltpu.sync_copy(r_mfi, out_mfi.at[pl.ds(off, ch)])
      pltpu.sync_copy(r_tu, out_tu.at[pl.ds(off, ch)])
      pltpu.sync_copy(r_ti, out_ti.at[pl.ds(off, ch)])

  return gather_kernel(user, item, mfu_lin, mfi_lin, tu_lin, ti_lin)


def _tc_head_body(mfu_ref, mfi_ref, tu_ref, ti_ref,
                  b1_ref, w2_ref, b2_ref, w3_ref, b3_ref,
                  wl_mf_ref, wl_mlp_ref, bl_ref, out_ref):
  f32 = jnp.float32
  h = jnp.maximum(tu_ref[...] + ti_ref[...] + b1_ref[...], 0.0)
  h = jnp.maximum(
      jnp.dot(h, w2_ref[...], preferred_element_type=f32) + b2_ref[...], 0.0)
  h = jnp.maximum(
      jnp.dot(h, w3_ref[...], preferred_element_type=f32) + b3_ref[...], 0.0)
  mf = mfu_ref[...] * mfi_ref[...]
  logit = (jnp.sum(mf * wl_mf_ref[...], axis=1)
           + jnp.sum(h * wl_mlp_ref[...], axis=1)
           + bl_ref[0, 0])
  out_ref[...] = jax.nn.sigmoid(logit)


def _tc_head(mfu, mfi, tu, ti, b1, W2, b2, W3, b3, Wl, bl):
  blk = 2048
  grid = (B // blk,)
  f32 = jnp.float32
  wl_mf = Wl[:MF_D, 0].reshape(1, MF_D)
  wl_mlp = Wl[MF_D:, 0].reshape(1, Wl.shape[0] - MF_D)
  b1r = b1.reshape(1, -1)
  b2r = b2.reshape(1, -1)
  b3r = b3.reshape(1, -1)
  blr = bl.reshape(1, 1)

  def rows_spec(d):
    return pl.BlockSpec((blk, d), lambda i: (i, 0))

  def full_spec(a):
    return pl.BlockSpec(a.shape, lambda i: tuple(0 for _ in a.shape))

  return pl.pallas_call(
      _tc_head_body,
      grid=grid,
      in_specs=[
          rows_spec(MF_D), rows_spec(MF_D), rows_spec(MLP_D), rows_spec(MLP_D),
          full_spec(b1r), full_spec(W2), full_spec(b2r),
          full_spec(W3), full_spec(b3r), full_spec(wl_mf), full_spec(wl_mlp),
          full_spec(blr),
      ],
      out_specs=pl.BlockSpec((blk,), lambda i: (i,)),
      out_shape=jax.ShapeDtypeStruct((B,), f32),
  )(mfu, mfi, tu, ti, b1r, W2, b2r, W3, b3r, wl_mf, wl_mlp, blr)


def kernel(user, item, mf_user_table, mf_item_table, mlp_user_table,
           mlp_item_table, W1, b1, W2, b2, W3, b3, Wl, bl):
  user = user.astype(jnp.int32)
  item = item.astype(jnp.int32)
  mfu_lin, mfi_lin, tu_lin, ti_lin = _tc_transform(
      mf_user_table, mf_item_table, mlp_user_table, mlp_item_table, W1)
  mfu, mfi, tu, ti = _sc_gather(user, item, mfu_lin, mfi_lin, tu_lin, ti_lin)
  return _tc_head(mfu, mfi, tu, ti, b1, W2, b2, W3, b3, Wl, bl)


# speedup vs baseline: 4.2289x; 4.2289x over previous
import functools

import jax
import jax.numpy as jnp
from jax import lax
from jax.experimental import pallas as pl
from jax.experimental.pallas import tpu as pltpu
from jax.experimental.pallas import tpu_sc as plsc

B = 16384
N_ROWS = 1000000
MF_D = 8
MLP_D = 32

_E = 2048
_G = pl.cdiv(N_ROWS, _E)
_N_PAD = _G * _E


def _t128(x):
  return jnp.transpose(x, (1, 0))


def _tc_transform_body(mfu_t, mfi_t, mlu_t, mli_t, w1u, w1i,
                       out_mfu, out_mfi, out_tu, out_ti, x8, x32):
  f32 = jnp.float32
  dn = (((0,), (0,)), ((), ()))
  for src, dst in ((mfu_t, out_mfu), (mfi_t, out_mfi)):
    for t in range(16):
      x8[pl.ds(t * MF_D, MF_D), :] = src[:, pl.ds(t * 128, 128)]
    dst[...] = _t128(x8[...])
  for src, w, dst in ((mlu_t, w1u, out_tu), (mli_t, w1i, out_ti)):
    a = lax.dot_general(w[...], src[...], dn, preferred_element_type=f32)
    for grp in range(4):
      for t in range(4):
        x32[pl.ds(t * MLP_D, MLP_D), :] = lax.slice(
            a, (0, grp * 512 + t * 128), (MLP_D, grp * 512 + (t + 1) * 128))
      dst[pl.ds(grp * 128, 128), :] = _t128(x32[...])


def _tc_transform(mf_user_table, mf_item_table, mlp_user_table,
                  mlp_item_table, W1):
  f32 = jnp.float32
  mfu_t = mf_user_table.T
  mfi_t = mf_item_table.T
  mlu_t = mlp_user_table.T
  mli_t = mlp_item_table.T
  w1u = W1[:MLP_D]
  w1i = W1[MLP_D:]

  def col_spec(d):
    return pl.BlockSpec((d, _E), lambda i: (0, i))

  def full_spec(a):
    return pl.BlockSpec(a.shape, lambda i: (0, 0))

  def packed_spec(d):
    return pl.BlockSpec((_E * d // 128, 128), lambda i: (i, 0))

  mfu_p, mfi_p, tu_p, ti_p = pl.pallas_call(
      _tc_transform_body,
      grid=(_G,),
      in_specs=[
          col_spec(MF_D), col_spec(MF_D), col_spec(MLP_D), col_spec(MLP_D),
          full_spec(w1u), full_spec(w1i),
      ],
      out_specs=[packed_spec(MF_D), packed_spec(MF_D), packed_spec(MLP_D),
                 packed_spec(MLP_D)],
      out_shape=[
          jax.ShapeDtypeStruct((_N_PAD * MF_D // 128, 128), f32),
          jax.ShapeDtypeStruct((_N_PAD * MF_D // 128, 128), f32),
          jax.ShapeDtypeStruct((_N_PAD * MLP_D // 128, 128), f32),
          jax.ShapeDtypeStruct((_N_PAD * MLP_D // 128, 128), f32),
      ],
      scratch_shapes=[
          pltpu.VMEM((128, 128), f32),
          pltpu.VMEM((128, 128), f32),
      ],
  )(mfu_t, mfi_t, mlu_t, mli_t, w1u, w1i)
  return (mfu_p.reshape(_N_PAD, MF_D), mfi_p.reshape(_N_PAD, MF_D),
          tu_p.reshape(_N_PAD, MLP_D), ti_p.reshape(_N_PAD, MLP_D))


def _sc_gather(user, item, mfu_lin, mfi_lin, tu_lin, ti_lin):
  info = plsc.get_sparse_core_info()
  nw = info.num_cores * info.num_subcores
  b_per_w = B // nw
  ch = 128
  n_chunks = b_per_w // ch
  mesh = plsc.VectorSubcoreMesh(core_axis_name="c", subcore_axis_name="s")
  f32 = jnp.float32
  i32 = jnp.int32

  @functools.partial(
      pl.kernel,
      mesh=mesh,
      compiler_params=pltpu.CompilerParams(use_tc_tiling_on_sc=False),
      out_type=[
          jax.ShapeDtypeStruct((B, MF_D), f32),
          jax.ShapeDtypeStruct((B, MF_D), f32),
          jax.ShapeDtypeStruct((B, MLP_D), f32),
          jax.ShapeDtypeStruct((B, MLP_D), f32),
      ],
      scratch_types=[
          pltpu.VMEM((ch,), i32),
          pltpu.VMEM((ch,), i32),
          pltpu.VMEM((ch,), i32),
          pltpu.VMEM((ch,), i32),
          pltpu.VMEM((ch, MF_D), f32),
          pltpu.VMEM((ch, MF_D), f32),
          pltpu.VMEM((ch, MLP_D), f32),
          pltpu.VMEM((ch, MLP_D), f32),
          pltpu.SemaphoreType.DMA,
      ],
  )
  def gather_kernel(user_h, item_h, mfu_h, mfi_h, tu_h, ti_h,
                    out_mfu, out_mfi, out_tu, out_ti,
                    idx8_u, idx8_i, idx32_u, idx32_i,
                    r_mfu, r_mfi, r_tu, r_ti, sem):
    wid = lax.axis_index("s") * info.num_cores + lax.axis_index("c")
    base = wid * b_per_w
    for c in range(n_chunks):
      off = base + c * ch
      pltpu.sync_copy(user_h.at[pl.ds(off, ch)], idx8_u)
      pltpu.sync_copy(item_h.at[pl.ds(off, ch)], idx8_i)
      for v in range(ch // 16):
        s = pl.ds(v * 16, 16)
        for src in (idx8_u, idx8_i):
          e = src[s]
          l = lax.bitwise_and(e, 127)
          t = lax.bitwise_and(lax.shift_right_logical(e, 7), 15)
          g8 = lax.bitwise_and(e, ~(_E - 1))
          pi8 = g8 + lax.shift_left(l, 4) + t
          t4 = lax.bitwise_and(lax.shift_right_logical(e, 7), 3)
          g32 = lax.bitwise_and(e, ~511)
          pi32 = g32 + lax.shift_left(l, 2) + t4
          if src is idx8_u:
            idx32_u[s] = pi32
            idx8_u[s] = pi8
          else:
            idx32_i[s] = pi32
            idx8_i[s] = pi8
      g1 = pltpu.async_copy(mfu_h.at[idx8_u], r_mfu, sem)
      g2 = pltpu.async_copy(mfi_h.at[idx8_i], r_mfi, sem)
      g3 = pltpu.async_copy(tu_h.at[idx32_u], r_tu, sem)
      g4 = pltpu.async_copy(ti_h.at[idx32_i], r_ti, sem)
      g1.wait()
      g2.wait()
      g3.wait()
      g4.wait()
      pltpu.sync_copy(r_mfu, out_mfu.at[pl.ds(off, ch)])
      pltpu.sync_copy(r_mfi, out_mfi.at[pl.ds(off, ch)])
      pltpu.sync_copy(r_tu, out_tu.at[pl.ds(off, ch)])
      pltpu.sync_copy(r_ti, out_ti.at[pl.ds(off, ch)])

  return gather_kernel(user, item, mfu_lin, mfi_lin, tu_lin, ti_lin)


def _tc_head_body(mfu_ref, mfi_ref, tu_ref, ti_ref,
                  b1_ref, w2_ref, b2_ref, w3_ref, b3_ref,
                  wl_mf_ref, wl_mlp_ref, bl_ref, out_ref):
  f32 = jnp.float32
  h = jnp.maximum(tu_ref[...] + ti_ref[...] + b1_ref[...], 0.0)
  h = jnp.maximum(
      jnp.dot(h, w2_ref[...], preferred_element_type=f32) + b2_ref[...], 0.0)
  h = jnp.maximum(
      jnp.dot(h, w3_ref[...], preferred_element_type=f32) + b3_ref[...], 0.0)
  mf = mfu_ref[...] * mfi_ref[...]
  logit = (jnp.sum(mf * wl_mf_ref[...], axis=1)
           + jnp.sum(h * wl_mlp_ref[...], axis=1)
           + bl_ref[0, 0])
  out_ref[...] = jax.nn.sigmoid(logit)


def _tc_head(mfu, mfi, tu, ti, b1, W2, b2, W3, b3, Wl, bl):
  blk = 2048
  grid = (B // blk,)
  f32 = jnp.float32
  wl_mf = Wl[:MF_D, 0].reshape(1, MF_D)
  wl_mlp = Wl[MF_D:, 0].reshape(1, Wl.shape[0] - MF_D)
  b1r = b1.reshape(1, -1)
  b2r = b2.reshape(1, -1)
  b3r = b3.reshape(1, -1)
  blr = bl.reshape(1, 1)

  def rows_spec(d):
    return pl.BlockSpec((blk, d), lambda i: (i, 0))

  def full_spec(a):
    return pl.BlockSpec(a.shape, lambda i: tuple(0 for _ in a.shape))

  return pl.pallas_call(
      _tc_head_body,
      grid=grid,
      in_specs=[
          rows_spec(MF_D), rows_spec(MF_D), rows_spec(MLP_D), rows_spec(MLP_D),
          full_spec(b1r), full_spec(W2), full_spec(b2r),
          full_spec(W3), full_spec(b3r), full_spec(wl_mf), full_spec(wl_mlp),
          full_spec(blr),
      ],
      out_specs=pl.BlockSpec((blk,), lambda i: (i,)),
      out_shape=jax.ShapeDtypeStruct((B,), f32),
  )(mfu, mfi, tu, ti, b1r, W2, b2r, W3, b3r, wl_mf, wl_mlp, blr)


def kernel(user, item, mf_user_table, mf_item_table, mlp_user_table,
           mlp_item_table, W1, b1, W2, b2, W3, b3, Wl, bl):
  user = user.astype(jnp.int32)
  item = item.astype(jnp.int32)
  mfu_lin, mfi_lin, tu_lin, ti_lin = _tc_transform(
      mf_user_table, mf_item_table, mlp_user_table, mlp_item_table, W1)
  mfu, mfi, tu, ti = _sc_gather(user, item, mfu_lin, mfi_lin, tu_lin, ti_lin)
  return _tc_head(mfu, mfi, tu, ti, b1, W2, b2, W3, b3, Wl, bl)


# speedup vs baseline: 8.2293x; 1.9460x over previous
import functools

import jax
import jax.numpy as jnp
from jax import lax
from jax.experimental import pallas as pl
from jax.experimental.pallas import tpu as pltpu
from jax.experimental.pallas import tpu_sc as plsc

B = 16384
N_ROWS = 1000000
MF_D = 8
MLP_D = 32

_E = 16384
_G = pl.cdiv(N_ROWS, _E)
_N_PAD = _G * _E


def _t128(x):
  return jnp.transpose(x, (1, 0))


def _tc_transform_body(mfu_t, mfi_t, mlu_t, mli_t, w1u, w1i,
                       out_mfu, out_mfi, out_tu, out_ti):
  f32 = jnp.float32
  dn = (((0,), (0,)), ((), ()))
  for src, dst in ((mfu_t, out_mfu), (mfi_t, out_mfi)):
    v = src[...]
    for grp in range(_E // 2048):
      x = jnp.concatenate(
          [lax.slice(v, (0, grp * 2048 + t * 128),
                     (MF_D, grp * 2048 + (t + 1) * 128)) for t in range(16)],
          axis=0)
      dst[pl.ds(grp * 128, 128), :] = _t128(x)
  for src, w, dst in ((mlu_t, w1u, out_tu), (mli_t, w1i, out_ti)):
    a = lax.dot_general(w[...], src[...], dn, preferred_element_type=f32)
    for grp in range(_E // 512):
      x = jnp.concatenate(
          [lax.slice(a, (0, grp * 512 + t * 128),
                     (MLP_D, grp * 512 + (t + 1) * 128)) for t in range(4)],
          axis=0)
      dst[pl.ds(grp * 128, 128), :] = _t128(x)


def _tc_transform(mf_user_table, mf_item_table, mlp_user_table,
                  mlp_item_table, W1):
  f32 = jnp.float32
  mfu_t = mf_user_table.T
  mfi_t = mf_item_table.T
  mlu_t = mlp_user_table.T
  mli_t = mlp_item_table.T
  w1u = W1[:MLP_D]
  w1i = W1[MLP_D:]

  def col_spec(d):
    return pl.BlockSpec((d, _E), lambda i: (0, i))

  def full_spec(a):
    return pl.BlockSpec(a.shape, lambda i: (0, 0))

  def packed_spec(d):
    return pl.BlockSpec((_E * d // 128, 128), lambda i: (i, 0))

  mfu_p, mfi_p, tu_p, ti_p = pl.pallas_call(
      _tc_transform_body,
      grid=(_G,),
      in_specs=[
          col_spec(MF_D), col_spec(MF_D), col_spec(MLP_D), col_spec(MLP_D),
          full_spec(w1u), full_spec(w1i),
      ],
      out_specs=[packed_spec(MF_D), packed_spec(MF_D), packed_spec(MLP_D),
                 packed_spec(MLP_D)],
      out_shape=[
          jax.ShapeDtypeStruct((_N_PAD * MF_D // 128, 128), f32),
          jax.ShapeDtypeStruct((_N_PAD * MF_D // 128, 128), f32),
          jax.ShapeDtypeStruct((_N_PAD * MLP_D // 128, 128), f32),
          jax.ShapeDtypeStruct((_N_PAD * MLP_D // 128, 128), f32),
      ],
  )(mfu_t, mfi_t, mlu_t, mli_t, w1u, w1i)
  return (mfu_p.reshape(_N_PAD, MF_D), mfi_p.reshape(_N_PAD, MF_D),
          tu_p.reshape(_N_PAD, MLP_D), ti_p.reshape(_N_PAD, MLP_D))


def _sc_gather(user, item, mfu_lin, mfi_lin, tu_lin, ti_lin):
  info = plsc.get_sparse_core_info()
  nw = info.num_cores * info.num_subcores
  b_per_w = B // nw
  ch = 128
  n_chunks = b_per_w // ch
  mesh = plsc.VectorSubcoreMesh(core_axis_name="c", subcore_axis_name="s")
  f32 = jnp.float32
  i32 = jnp.int32

  @functools.partial(
      pl.kernel,
      mesh=mesh,
      compiler_params=pltpu.CompilerParams(use_tc_tiling_on_sc=False),
      out_type=[
          jax.ShapeDtypeStruct((B, MF_D), f32),
          jax.ShapeDtypeStruct((B, MF_D), f32),
          jax.ShapeDtypeStruct((B, MLP_D), f32),
          jax.ShapeDtypeStruct((B, MLP_D), f32),
      ],
      scratch_types=[
          pltpu.VMEM((ch,), i32),
          pltpu.VMEM((ch,), i32),
          pltpu.VMEM((ch,), i32),
          pltpu.VMEM((ch,), i32),
          pltpu.VMEM((ch, MF_D), f32),
          pltpu.VMEM((ch, MF_D), f32),
          pltpu.VMEM((ch, MLP_D), f32),
          pltpu.VMEM((ch, MLP_D), f32),
          pltpu.SemaphoreType.DMA,
      ],
  )
  def gather_kernel(user_h, item_h, mfu_h, mfi_h, tu_h, ti_h,
                    out_mfu, out_mfi, out_tu, out_ti,
                    idx8_u, idx8_i, idx32_u, idx32_i,
                    r_mfu, r_mfi, r_tu, r_ti, sem):
    wid = lax.axis_index("s") * info.num_cores + lax.axis_index("c")
    base = wid * b_per_w
    for c in range(n_chunks):
      off = base + c * ch
      pltpu.sync_copy(user_h.at[pl.ds(off, ch)], idx8_u)
      pltpu.sync_copy(item_h.at[pl.ds(off, ch)], idx8_i)
      for v in range(ch // 16):
        s = pl.ds(v * 16, 16)
        for src in (idx8_u, idx8_i):
          e = src[s]
          l = lax.bitwise_and(e, 127)
          t = lax.bitwise_and(lax.shift_right_logical(e, 7), 15)
          g8 = lax.bitwise_and(e, ~2047)
          pi8 = g8 + lax.shift_left(l, 4) + t
          t4 = lax.bitwise_and(lax.shift_right_logical(e, 7), 3)
          g32 = lax.bitwise_and(e, ~511)
          pi32 = g32 + lax.shift_left(l, 2) + t4
          if src is idx8_u:
            idx32_u[s] = pi32
            idx8_u[s] = pi8
          else:
            idx32_i[s] = pi32
            idx8_i[s] = pi8
      g1 = pltpu.async_copy(mfu_h.at[idx8_u], r_mfu, sem)
      g2 = pltpu.async_copy(mfi_h.at[idx8_i], r_mfi, sem)
      g3 = pltpu.async_copy(tu_h.at[idx32_u], r_tu, sem)
      g4 = pltpu.async_copy(ti_h.at[idx32_i], r_ti, sem)
      g1.wait()
      g2.wait()
      g3.wait()
      g4.wait()
      pltpu.sync_copy(r_mfu, out_mfu.at[pl.ds(off, ch)])
      pltpu.sync_copy(r_mfi, out_mfi.at[pl.ds(off, ch)])
      pltpu.sync_copy(r_tu, out_tu.at[pl.ds(off, ch)])
      pltpu.sync_copy(r_ti, out_ti.at[pl.ds(off, ch)])

  return gather_kernel(user, item, mfu_lin, mfi_lin, tu_lin, ti_lin)


def _tc_head_body(mfu_ref, mfi_ref, tu_ref, ti_ref,
                  b1_ref, w2_ref, b2_ref, w3_ref, b3_ref,
                  wl_mf_ref, wl_mlp_ref, bl_ref, out_ref):
  f32 = jnp.float32
  h = jnp.maximum(tu_ref[...] + ti_ref[...] + b1_ref[...], 0.0)
  h = jnp.maximum(
      jnp.dot(h, w2_ref[...], preferred_element_type=f32) + b2_ref[...], 0.0)
  h = jnp.maximum(
      jnp.dot(h, w3_ref[...], preferred_element_type=f32) + b3_ref[...], 0.0)
  mf = mfu_ref[...] * mfi_ref[...]
  logit = (jnp.sum(mf * wl_mf_ref[...], axis=1)
           + jnp.sum(h * wl_mlp_ref[...], axis=1)
           + bl_ref[0, 0])
  out_ref[...] = jax.nn.sigmoid(logit)


def _tc_head(mfu, mfi, tu, ti, b1, W2, b2, W3, b3, Wl, bl):
  blk = 2048
  grid = (B // blk,)
  f32 = jnp.float32
  wl_mf = Wl[:MF_D, 0].reshape(1, MF_D)
  wl_mlp = Wl[MF_D:, 0].reshape(1, Wl.shape[0] - MF_D)
  b1r = b1.reshape(1, -1)
  b2r = b2.reshape(1, -1)
  b3r = b3.reshape(1, -1)
  blr = bl.reshape(1, 1)

  def rows_spec(d):
    return pl.BlockSpec((blk, d), lambda i: (i, 0))

  def full_spec(a):
    return pl.BlockSpec(a.shape, lambda i: tuple(0 for _ in a.shape))

  return pl.pallas_call(
      _tc_head_body,
      grid=grid,
      in_specs=[
          rows_spec(MF_D), rows_spec(MF_D), rows_spec(MLP_D), rows_spec(MLP_D),
          full_spec(b1r), full_spec(W2), full_spec(b2r),
          full_spec(W3), full_spec(b3r), full_spec(wl_mf), full_spec(wl_mlp),
          full_spec(blr),
      ],
      out_specs=pl.BlockSpec((blk,), lambda i: (i,)),
      out_shape=jax.ShapeDtypeStruct((B,), f32),
  )(mfu, mfi, tu, ti, b1r, W2, b2r, W3, b3r, wl_mf, wl_mlp, blr)


def kernel(user, item, mf_user_table, mf_item_table, mlp_user_table,
           mlp_item_table, W1, b1, W2, b2, W3, b3, Wl, bl):
  user = user.astype(jnp.int32)
  item = item.astype(jnp.int32)
  mfu_lin, mfi_lin, tu_lin, ti_lin = _tc_transform(
      mf_user_table, mf_item_table, mlp_user_table, mlp_item_table, W1)
  mfu, mfi, tu, ti = _sc_gather(user, item, mfu_lin, mfi_lin, tu_lin, ti_lin)
  return _tc_head(mfu, mfi, tu, ti, b1, W2, b2, W3, b3, Wl, bl)


# speedup vs baseline: 8.4394x; 1.0255x over previous
import functools

import jax
import jax.numpy as jnp
from jax import lax
from jax.experimental import pallas as pl
from jax.experimental.pallas import tpu as pltpu
from jax.experimental.pallas import tpu_sc as plsc

B = 16384
N_ROWS = 1000000
MF_D = 8
MLP_D = 32

_E = 32768
_G = pl.cdiv(N_ROWS, _E)
_N_PAD = _G * _E


def _t128(x):
  return jnp.transpose(x, (1, 0))


def _tc_transform_body(mfu_t, mfi_t, mlu_t, mli_t, w1u, w1i,
                       out_mfu, out_mfi, out_tu, out_ti):
  f32 = jnp.float32
  dn = (((0,), (0,)), ((), ()))
  for src, dst in ((mfu_t, out_mfu), (mfi_t, out_mfi)):
    v = src[...]
    for grp in range(_E // 2048):
      x = jnp.concatenate(
          [lax.slice(v, (0, grp * 2048 + t * 128),
                     (MF_D, grp * 2048 + (t + 1) * 128)) for t in range(16)],
          axis=0)
      dst[pl.ds(grp * 128, 128), :] = _t128(x)
  for src, w, dst in ((mlu_t, w1u, out_tu), (mli_t, w1i, out_ti)):
    a = lax.dot_general(w[...], src[...], dn, preferred_element_type=f32)
    for grp in range(_E // 512):
      x = jnp.concatenate(
          [lax.slice(a, (0, grp * 512 + t * 128),
                     (MLP_D, grp * 512 + (t + 1) * 128)) for t in range(4)],
          axis=0)
      dst[pl.ds(grp * 128, 128), :] = _t128(x)


def _tc_transform(mf_user_table, mf_item_table, mlp_user_table,
                  mlp_item_table, W1):
  f32 = jnp.float32
  mfu_t = mf_user_table.T
  mfi_t = mf_item_table.T
  mlu_t = mlp_user_table.T
  mli_t = mlp_item_table.T
  w1u = W1[:MLP_D]
  w1i = W1[MLP_D:]

  def col_spec(d):
    return pl.BlockSpec((d, _E), lambda i: (0, i))

  def full_spec(a):
    return pl.BlockSpec(a.shape, lambda i: (0, 0))

  def packed_spec(d):
    return pl.BlockSpec((_E * d // 128, 128), lambda i: (i, 0))

  mfu_p, mfi_p, tu_p, ti_p = pl.pallas_call(
      _tc_transform_body,
      grid=(_G,),
      in_specs=[
          col_spec(MF_D), col_spec(MF_D), col_spec(MLP_D), col_spec(MLP_D),
          full_spec(w1u), full_spec(w1i),
      ],
      out_specs=[packed_spec(MF_D), packed_spec(MF_D), packed_spec(MLP_D),
                 packed_spec(MLP_D)],
      out_shape=[
          jax.ShapeDtypeStruct((_N_PAD * MF_D // 128, 128), f32),
          jax.ShapeDtypeStruct((_N_PAD * MF_D // 128, 128), f32),
          jax.ShapeDtypeStruct((_N_PAD * MLP_D // 128, 128), f32),
          jax.ShapeDtypeStruct((_N_PAD * MLP_D // 128, 128), f32),
      ],
  )(mfu_t, mfi_t, mlu_t, mli_t, w1u, w1i)
  return (mfu_p.reshape(_N_PAD, MF_D), mfi_p.reshape(_N_PAD, MF_D),
          tu_p.reshape(_N_PAD, MLP_D), ti_p.reshape(_N_PAD, MLP_D))


def _sc_gather(user, item, mfu_lin, mfi_lin, tu_lin, ti_lin):
  info = plsc.get_sparse_core_info()
  nw = info.num_cores * info.num_subcores
  b_per_w = B // nw
  ch = 128
  n_chunks = b_per_w // ch
  mesh = plsc.VectorSubcoreMesh(core_axis_name="c", subcore_axis_name="s")
  f32 = jnp.float32
  i32 = jnp.int32

  @functools.partial(
      pl.kernel,
      mesh=mesh,
      compiler_params=pltpu.CompilerParams(use_tc_tiling_on_sc=False),
      out_type=[
          jax.ShapeDtypeStruct((B, MF_D), f32),
          jax.ShapeDtypeStruct((B, MF_D), f32),
          jax.ShapeDtypeStruct((B, MLP_D), f32),
          jax.ShapeDtypeStruct((B, MLP_D), f32),
      ],
      scratch_types=[
          pltpu.VMEM((ch,), i32),
          pltpu.VMEM((ch,), i32),
          pltpu.VMEM((ch,), i32),
          pltpu.VMEM((ch,), i32),
          pltpu.VMEM((ch, MF_D), f32),
          pltpu.VMEM((ch, MF_D), f32),
          pltpu.VMEM((ch, MLP_D), f32),
          pltpu.VMEM((ch, MLP_D), f32),
          pltpu.SemaphoreType.DMA,
      ],
  )
  def gather_kernel(user_h, item_h, mfu_h, mfi_h, tu_h, ti_h,
                    out_mfu, out_mfi, out_tu, out_ti,
                    idx8_u, idx8_i, idx32_u, idx32_i,
                    r_mfu, r_mfi, r_tu, r_ti, sem):
    wid = lax.axis_index("s") * info.num_cores + lax.axis_index("c")
    base = wid * b_per_w
    for c in range(n_chunks):
      off = base + c * ch
      pltpu.sync_copy(user_h.at[pl.ds(off, ch)], idx8_u)
      pltpu.sync_copy(item_h.at[pl.ds(off, ch)], idx8_i)
      for v in range(ch // 16):
        s = pl.ds(v * 16, 16)
        for src in (idx8_u, idx8_i):
          e = src[s]
          l = lax.bitwise_and(e, 127)
          t = lax.bitwise_and(lax.shift_right_logical(e, 7), 15)
          g8 = lax.bitwise_and(e, ~2047)
          pi8 = g8 + lax.shift_left(l, 4) + t
          t4 = lax.bitwise_and(lax.shift_right_logical(e, 7), 3)
          g32 = lax.bitwise_and(e, ~511)
          pi32 = g32 + lax.shift_left(l, 2) + t4
          if src is idx8_u:
            idx32_u[s] = pi32
            idx8_u[s] = pi8
          else:
            idx32_i[s] = pi32
            idx8_i[s] = pi8
      g1 = pltpu.async_copy(mfu_h.at[idx8_u], r_mfu, sem)
      g2 = pltpu.async_copy(mfi_h.at[idx8_i], r_mfi, sem)
      g3 = pltpu.async_copy(tu_h.at[idx32_u], r_tu, sem)
      g4 = pltpu.async_copy(ti_h.at[idx32_i], r_ti, sem)
      g1.wait()
      g2.wait()
      g3.wait()
      g4.wait()
      pltpu.sync_copy(r_mfu, out_mfu.at[pl.ds(off, ch)])
      pltpu.sync_copy(r_mfi, out_mfi.at[pl.ds(off, ch)])
      pltpu.sync_copy(r_tu, out_tu.at[pl.ds(off, ch)])
      pltpu.sync_copy(r_ti, out_ti.at[pl.ds(off, ch)])

  return gather_kernel(user, item, mfu_lin, mfi_lin, tu_lin, ti_lin)


def _tc_head_body(mfu_ref, mfi_ref, tu_ref, ti_ref,
                  b1_ref, w2_ref, b2_ref, w3_ref, b3_ref,
                  wl_mf_ref, wl_mlp_ref, bl_ref, out_ref):
  f32 = jnp.float32
  h = jnp.maximum(tu_ref[...] + ti_ref[...] + b1_ref[...], 0.0)
  h = jnp.maximum(
      jnp.dot(h, w2_ref[...], preferred_element_type=f32) + b2_ref[...], 0.0)
  h = jnp.maximum(
      jnp.dot(h, w3_ref[...], preferred_element_type=f32) + b3_ref[...], 0.0)
  mf = mfu_ref[...] * mfi_ref[...]
  logit = (jnp.sum(mf * wl_mf_ref[...], axis=1)
           + jnp.sum(h * wl_mlp_ref[...], axis=1)
           + bl_ref[0, 0])
  out_ref[...] = jax.nn.sigmoid(logit)


def _tc_head(mfu, mfi, tu, ti, b1, W2, b2, W3, b3, Wl, bl):
  blk = 2048
  grid = (B // blk,)
  f32 = jnp.float32
  wl_mf = Wl[:MF_D, 0].reshape(1, MF_D)
  wl_mlp = Wl[MF_D:, 0].reshape(1, Wl.shape[0] - MF_D)
  b1r = b1.reshape(1, -1)
  b2r = b2.reshape(1, -1)
  b3r = b3.reshape(1, -1)
  blr = bl.reshape(1, 1)

  def rows_spec(d):
    return pl.BlockSpec((blk, d), lambda i: (i, 0))

  def full_spec(a):
    return pl.BlockSpec(a.shape, lambda i: tuple(0 for _ in a.shape))

  return pl.pallas_call(
      _tc_head_body,
      grid=grid,
      in_specs=[
          rows_spec(MF_D), rows_spec(MF_D), rows_spec(MLP_D), rows_spec(MLP_D),
          full_spec(b1r), full_spec(W2), full_spec(b2r),
          full_spec(W3), full_spec(b3r), full_spec(wl_mf), full_spec(wl_mlp),
          full_spec(blr),
      ],
      out_specs=pl.BlockSpec((blk,), lambda i: (i,)),
      out_shape=jax.ShapeDtypeStruct((B,), f32),
  )(mfu, mfi, tu, ti, b1r, W2, b2r, W3, b3r, wl_mf, wl_mlp, blr)


def kernel(user, item, mf_user_table, mf_item_table, mlp_user_table,
           mlp_item_table, W1, b1, W2, b2, W3, b3, Wl, bl):
  user = user.astype(jnp.int32)
  item = item.astype(jnp.int32)
  mfu_lin, mfi_lin, tu_lin, ti_lin = _tc_transform(
      mf_user_table, mf_item_table, mlp_user_table, mlp_item_table, W1)
  mfu, mfi, tu, ti = _sc_gather(user, item, mfu_lin, mfi_lin, tu_lin, ti_lin)
  return _tc_head(mfu, mfi, tu, ti, b1, W2, b2, W3, b3, Wl, bl)
